# baseline (speedup 1.0000x reference)
import jax
import jax.numpy as jnp
from jax import lax
from jax.experimental import pallas as pl
from jax.experimental.pallas import tpu as pltpu
from jax.experimental.pallas import tpu_sc as plsc

N = 10000
E = 320000
IN_CH = 128
HID = 16
OUT_CH = 128

NC, NS = 2, 16
NW = NC * NS
CHUNK = 128
NCH = E // CHUNK
CPT = NCH // NW
NEXTRA = NCH - CPT * NW
NBUF = 6
ROUNDS = CPT // NBUF
SLAB = 632
NPAD = NS * SLAB
TSLAB = N // NS
OSLAB = TSLAB

_mesh = plsc.VectorSubcoreMesh(core_axis_name="c", subcore_axis_name="s")
_sc_params = pltpu.CompilerParams(use_tc_tiling_on_sc=False)
_f32 = jnp.float32


def _edge_pipeline(tab, sd_v, acc, bufs, gsems, ssems):
    def gather(j, b):
        pltpu.async_copy(tab.at[sd_v.at[j, 0]], bufs[b], gsems[b])

    def gather_wait(b):
        pltpu.make_async_copy(tab.at[sd_v.at[0, 0]], bufs[b], gsems[b]).wait()

    def scatter(j, b):
        pltpu.async_copy(bufs[b], acc.at[sd_v.at[j, 1]], ssems[b], add=True)

    def scatter_wait(b):
        pltpu.make_async_copy(bufs[b], acc.at[sd_v.at[0, 1]], ssems[b]).wait()

    for b in range(NBUF):
        gather(b, b)

    def round_body(g, carry):
        for b in range(NBUF):
            gather_wait(b)
            scatter(g * NBUF + b, b)
        for b in range(NBUF):
            scatter_wait(b)
            gather((g + 1) * NBUF + b, b)
        return carry

    lax.fori_loop(0, ROUNDS - 1, round_body, 0)
    g = ROUNDS - 1
    for b in range(NBUF):
        gather_wait(b)
        scatter(g * NBUF + b, b)
    for b in range(NBUF):
        scatter_wait(b)


def _sc_segsum_body(table, edges, zeros,
                    out,
                    sd_v, sdx, b0, b1, b2, b3, b4, b5, tab, acc,
                    gs0, gs1, gs2, gs3, gs4, gs5,
                    ss0, ss1, ss2, ss3, ss4, ss5):
    cid = lax.axis_index("c")
    sid = lax.axis_index("s")
    wid = cid * NS + sid
    pltpu.sync_copy(table.at[pl.ds(sid * TSLAB, TSLAB)],
                    tab.at[pl.ds(sid * TSLAB, TSLAB)])
    pltpu.sync_copy(zeros, acc.at[pl.ds(sid * SLAB, SLAB)])
    pltpu.sync_copy(edges.at[pl.ds(wid * CPT, CPT)], sd_v)
    plsc.subcore_barrier()

    @pl.when(wid < NEXTRA)
    def _():
        pltpu.sync_copy(edges.at[pl.ds(CPT * NW + wid, 1)], sdx)
        pltpu.async_copy(tab.at[sdx.at[0, 0]], b0, gs0).wait()
        pltpu.async_copy(b0, acc.at[sdx.at[0, 1]], ss0, add=True).wait()

    _edge_pipeline(tab, sd_v, acc,
                   [b0, b1, b2, b3, b4, b5],
                   [gs0, gs1, gs2, gs3, gs4, gs5],
                   [ss0, ss1, ss2, ss3, ss4, ss5])
    plsc.subcore_barrier()
    pltpu.sync_copy(acc.at[pl.ds(sid * TSLAB, TSLAB)],
                    out.at[cid, pl.ds(sid * TSLAB, TSLAB)])


def _sc_cnt_body(edges, zeros, ones_h,
                 cnt_out,
                 sd_v, sdx, ones_v, cnt_acc,
                 cs0, cs1, cs2, cs3, cs4, cs5):
    cid = lax.axis_index("c")
    sid = lax.axis_index("s")
    wid = cid * NS + sid
    csems = [cs0, cs1, cs2, cs3, cs4, cs5]
    pltpu.sync_copy(zeros, cnt_acc.at[pl.ds(sid * SLAB, SLAB)])
    pltpu.sync_copy(ones_h, ones_v)
    pltpu.sync_copy(edges.at[pl.ds(wid * CPT, CPT)], sd_v)
    plsc.subcore_barrier()

    def scatter(j, b):
        pltpu.async_copy(ones_v, cnt_acc.at[sd_v.at[j, 1]], csems[b],
                         add=True)

    def scatter_wait(b):
        pltpu.make_async_copy(ones_v, cnt_acc.at[sd_v.at[0, 1]],
                              csems[b]).wait()

    @pl.when(wid < NEXTRA)
    def _():
        pltpu.sync_copy(edges.at[pl.ds(CPT * NW + wid, 1)], sdx)
        pltpu.async_copy(ones_v, cnt_acc.at[sdx.at[0, 1]], cs0,
                         add=True).wait()

    for b in range(NBUF):
        scatter(b, b)

    def round_body(g, carry):
        for b in range(NBUF):
            scatter_wait(b)
            scatter((g + 1) * NBUF + b, b)
        return carry

    lax.fori_loop(0, ROUNDS - 1, round_body, 0)
    for b in range(NBUF):
        scatter_wait(b)
    plsc.subcore_barrier()
    pltpu.sync_copy(cnt_acc.at[pl.ds(sid * TSLAB, TSLAB)],
                    cnt_out.at[cid, pl.ds(sid * TSLAB, TSLAB)])


_DMA = pltpu.SemaphoreType.DMA

_cnt = pl.kernel(
    _sc_cnt_body,
    out_type=jax.ShapeDtypeStruct((NC, N, HID), _f32),
    mesh=_mesh,
    scratch_types=(
        [pltpu.VMEM((CPT, 2, CHUNK), jnp.int32),
         pltpu.VMEM((1, 2, CHUNK), jnp.int32),
         pltpu.VMEM((CHUNK, HID), _f32),
         pltpu.VMEM_SHARED((NPAD, HID), _f32)]
        + [_DMA] * NBUF
    ),
    compiler_params=_sc_params,
)

_seg = pl.kernel(
    _sc_segsum_body,
    out_type=jax.ShapeDtypeStruct((NC, N, HID), _f32),
    mesh=_mesh,
    scratch_types=(
        [pltpu.VMEM((CPT, 2, CHUNK), jnp.int32),
         pltpu.VMEM((1, 2, CHUNK), jnp.int32)]
        + [pltpu.VMEM((CHUNK, HID), _f32)] * NBUF
        + [pltpu.VMEM_SHARED((NPAD, HID), _f32)] * 2
        + [_DMA] * (2 * NBUF)
    ),
    compiler_params=_sc_params,
)


PK = 8
NR = N // PK


def _proj_body(x3_ref, wl_ref, wr_ref, p_ref, r_ref):
    x3 = x3_ref[...]
    wl = wl_ref[...]
    wr = wr_ref[...]
    pparts, rparts = [], []
    for b in range(PK):
        xb = x3[:, b, :]
        pparts.append(lax.dot_general(xb, wl, (((1,), (1,)), ((), ())),
                                      preferred_element_type=_f32))
        rparts.append(lax.dot_general(xb, wr, (((1,), (1,)), ((), ())),
                                      preferred_element_type=_f32))
    p_ref[...] = jnp.concatenate(pparts, axis=1)
    r_ref[...] = jnp.concatenate(rparts, axis=1)


def _mid_body(s_ref, c_ref, bl_ref, r_ref,
              wl_ref, wr_ref, p_out, r_out, inv_out):
    inv = 1.0 / jnp.maximum(c_ref[0] + c_ref[1], 1.0)
    h = jnp.maximum((s_ref[0] + s_ref[1]) * inv
                    + bl_ref[...] + r_ref[...], 0.0)
    p_out[...] = jnp.dot(h, wl_ref[...], preferred_element_type=_f32)
    r_out[...] = jnp.dot(h, wr_ref[...], preferred_element_type=_f32)
    inv_out[...] = inv


def _act_body(s_ref, inv_ref, bl_ref, r_ref, h_out):
    h_out[...] = jnp.maximum((s_ref[0] + s_ref[1]) * inv_ref[...]
                             + bl_ref[...] + r_ref[...], 0.0)


def _final_body(s_ref, inv_ref, h_ref, wl_ref, wr_ref, bl_ref, out_ref):
    mean = (s_ref[0] + s_ref[1]) * inv_ref[...]
    out_ref[...] = (jnp.dot(mean, wl_ref[...], preferred_element_type=_f32)
                    + jnp.dot(h_ref[...], wr_ref[...],
                              preferred_element_type=_f32)
                    + bl_ref[...])


_proj = pl.pallas_call(
    _proj_body,
    out_shape=(jax.ShapeDtypeStruct((NR, 128), _f32),
               jax.ShapeDtypeStruct((NR, 128), _f32)))

_mid = pl.pallas_call(
    _mid_body,
    out_shape=(jax.ShapeDtypeStruct((NR, 128), _f32),
               jax.ShapeDtypeStruct((NR, 128), _f32),
               jax.ShapeDtypeStruct((NR, 128), _f32)))

_act = pl.pallas_call(
    _act_body,
    out_shape=jax.ShapeDtypeStruct((NR, 128), _f32))

_final = pl.pallas_call(
    _final_body,
    out_shape=jax.ShapeDtypeStruct((NR, PK * OUT_CH), _f32))


def kernel(edge_index, features, Wl0, bl0, Wr0, Wl1, bl1, Wr1, Wl2, bl2, Wr2):
    edges_t = edge_index.astype(jnp.int32).reshape(2, NCH, CHUNK).transpose(
        1, 0, 2)
    zeros = jnp.zeros((SLAB, HID), _f32)
    ones = jnp.ones((CHUNK, HID), _f32)
    x3 = features.reshape(NR, PK, IN_CH)
    eye8 = jnp.eye(PK, dtype=_f32)
    bd1l = jnp.kron(eye8, Wl1.T)
    bd1r = jnp.kron(eye8, Wr1.T)
    ws2l = jnp.kron(eye8, Wl2.T)
    ws2r = jnp.kron(eye8, Wr2.T)
    blt0 = jnp.tile(bl0, PK).reshape(1, 128)
    blt1 = jnp.tile(bl1, PK).reshape(1, 128)
    blt2 = jnp.tile(bl2, PK).reshape(1, PK * OUT_CH)

    c0 = _cnt(edges_t, zeros, ones)
    p0p, r0p = _proj(x3, Wl0, Wr0)
    s0 = _seg(p0p.reshape(N, HID), edges_t, zeros)
    p1p, r1p, invp = _mid(s0.reshape(NC, NR, 128), c0.reshape(NC, NR, 128),
                          blt0, r0p, bd1l, bd1r)
    s1 = _seg(p1p.reshape(N, HID), edges_t, zeros)
    h1p = _act(s1.reshape(NC, NR, 128), invp, blt1, r1p)
    s2 = _seg(h1p.reshape(N, HID), edges_t, zeros)
    outb = _final(s2.reshape(NC, NR, 128), invp, h1p, ws2l, ws2r, blt2)
    return outb.reshape(N, OUT_CH)


# speedup vs baseline: 1.0151x; 1.0151x over previous
import jax
import jax.numpy as jnp
from jax import lax
from jax.experimental import pallas as pl
from jax.experimental.pallas import tpu as pltpu
from jax.experimental.pallas import tpu_sc as plsc

N = 10000
E = 320000
IN_CH = 128
HID = 16
OUT_CH = 128

NC, NS = 2, 16
NW = NC * NS
CHUNK = 128
NCH = E // CHUNK
CPT = NCH // NW
NEXTRA = NCH - CPT * NW
NBUF = 6
ROUNDS = CPT // NBUF
SLAB = 632
NPAD = NS * SLAB
TSLAB = N // NS
OSLAB = TSLAB

_mesh = plsc.VectorSubcoreMesh(core_axis_name="c", subcore_axis_name="s")
_sc_params = pltpu.CompilerParams(use_tc_tiling_on_sc=False)
_f32 = jnp.float32


def _edge_pipeline(tab, sd_v, acc, bufs, gsems, ssems,
                   ones_v=None, cnt_acc=None, csems=None):
    def gather(j, b):
        pltpu.async_copy(tab.at[sd_v.at[j, 0]], bufs[b], gsems[b])

    def gather_wait(b):
        pltpu.make_async_copy(tab.at[sd_v.at[0, 0]], bufs[b], gsems[b]).wait()

    def scatter(j, b):
        pltpu.async_copy(bufs[b], acc.at[sd_v.at[j, 1]], ssems[b], add=True)

    def scatter_wait(b):
        pltpu.make_async_copy(bufs[b], acc.at[sd_v.at[0, 1]], ssems[b]).wait()

    def cnt_wait(b):
        pltpu.make_async_copy(ones_v, cnt_acc.at[sd_v.at[0, 1]],
                              csems[b]).wait()

    for b in range(NBUF):
        gather(b, b)

    def round_body(g, carry):
        for b in range(NBUF):
            j = g * NBUF + b
            gather_wait(b)
            scatter(j, b)
            if cnt_acc is not None:
                @pl.when(g > 0)
                def _():
                    cnt_wait(b)
                pltpu.async_copy(ones_v, cnt_acc.at[sd_v.at[j, 1]], csems[b],
                                 add=True)
        for b in range(NBUF):
            scatter_wait(b)
            gather((g + 1) * NBUF + b, b)
        return carry

    lax.fori_loop(0, ROUNDS - 1, round_body, 0)
    g = ROUNDS - 1
    for b in range(NBUF):
        j = g * NBUF + b
        gather_wait(b)
        scatter(j, b)
        if cnt_acc is not None:
            cnt_wait(b)
            pltpu.async_copy(ones_v, cnt_acc.at[sd_v.at[j, 1]], csems[b],
                             add=True)
    for b in range(NBUF):
        scatter_wait(b)
        if cnt_acc is not None:
            cnt_wait(b)


def _extra_chunk(wid, edges, tab, acc, buf, gsem, ssem, sdx,
                 ones_v=None, cnt_acc=None, csem=None):
    @pl.when(wid < NEXTRA)
    def _():
        pltpu.sync_copy(edges.at[pl.ds(CPT * NW + wid, 1)], sdx)
        pltpu.async_copy(tab.at[sdx.at[0, 0]], buf, gsem).wait()
        pltpu.async_copy(buf, acc.at[sdx.at[0, 1]], ssem, add=True).wait()
        if cnt_acc is not None:
            pltpu.async_copy(ones_v, cnt_acc.at[sdx.at[0, 1]], csem,
                             add=True).wait()


def _sc_segsum_cnt_body(table, edges, zeros, ones_h,
                        out, cnt_out,
                        sd_v, sdx, ones_v,
                        b0, b1, b2, b3, b4, b5, tab, acc, cnt_acc,
                        gs0, gs1, gs2, gs3, gs4, gs5,
                        ss0, ss1, ss2, ss3, ss4, ss5,
                        cs0, cs1, cs2, cs3, cs4, cs5):
    cid = lax.axis_index("c")
    sid = lax.axis_index("s")
    wid = cid * NS + sid
    zsl = pl.ds(sid * SLAB, SLAB)
    tsl = pl.ds(sid * TSLAB, TSLAB)
    pltpu.sync_copy(table.at[tsl], tab.at[tsl])
    pltpu.sync_copy(zeros, acc.at[zsl])
    pltpu.sync_copy(zeros, cnt_acc.at[zsl])
    pltpu.sync_copy(ones_h, ones_v)
    pltpu.sync_copy(edges.at[pl.ds(wid * CPT, CPT)], sd_v)
    plsc.subcore_barrier()
    _extra_chunk(wid, edges, tab, acc, b0, gs0, ss0, sdx,
                 ones_v=ones_v, cnt_acc=cnt_acc, csem=cs0)
    _edge_pipeline(tab, sd_v, acc,
                   [b0, b1, b2, b3, b4, b5],
                   [gs0, gs1, gs2, gs3, gs4, gs5],
                   [ss0, ss1, ss2, ss3, ss4, ss5],
                   ones_v=ones_v, cnt_acc=cnt_acc,
                   csems=[cs0, cs1, cs2, cs3, cs4, cs5])
    plsc.subcore_barrier()
    pltpu.sync_copy(acc.at[tsl], out.at[cid, tsl])
    pltpu.sync_copy(cnt_acc.at[tsl], cnt_out.at[cid, tsl])


def _sc_segsum_body(table, edges, zeros,
                    out,
                    sd_v, sdx, b0, b1, b2, b3, b4, b5, tab, acc,
                    gs0, gs1, gs2, gs3, gs4, gs5,
                    ss0, ss1, ss2, ss3, ss4, ss5):
    cid = lax.axis_index("c")
    sid = lax.axis_index("s")
    wid = cid * NS + sid
    zsl = pl.ds(sid * SLAB, SLAB)
    tsl = pl.ds(sid * TSLAB, TSLAB)
    pltpu.sync_copy(table.at[tsl], tab.at[tsl])
    pltpu.sync_copy(zeros, acc.at[zsl])
    pltpu.sync_copy(edges.at[pl.ds(wid * CPT, CPT)], sd_v)
    plsc.subcore_barrier()
    _extra_chunk(wid, edges, tab, acc, b0, gs0, ss0, sdx)
    _edge_pipeline(tab, sd_v, acc,
                   [b0, b1, b2, b3, b4, b5],
                   [gs0, gs1, gs2, gs3, gs4, gs5],
                   [ss0, ss1, ss2, ss3, ss4, ss5])
    plsc.subcore_barrier()
    pltpu.sync_copy(acc.at[tsl], out.at[cid, tsl])


_DMA = pltpu.SemaphoreType.DMA

_seg_cnt = pl.kernel(
    _sc_segsum_cnt_body,
    out_type=(jax.ShapeDtypeStruct((NC, N, HID), _f32),
              jax.ShapeDtypeStruct((NC, N, HID), _f32)),
    mesh=_mesh,
    scratch_types=(
        [pltpu.VMEM((CPT, 2, CHUNK), jnp.int32),
         pltpu.VMEM((1, 2, CHUNK), jnp.int32)]
        + [pltpu.VMEM((CHUNK, HID), _f32)] * (1 + NBUF)
        + [pltpu.VMEM_SHARED((NPAD, HID), _f32)] * 3
        + [_DMA] * (3 * NBUF)
    ),
    compiler_params=_sc_params,
)

_seg = pl.kernel(
    _sc_segsum_body,
    out_type=jax.ShapeDtypeStruct((NC, N, HID), _f32),
    mesh=_mesh,
    scratch_types=(
        [pltpu.VMEM((CPT, 2, CHUNK), jnp.int32),
         pltpu.VMEM((1, 2, CHUNK), jnp.int32)]
        + [pltpu.VMEM((CHUNK, HID), _f32)] * NBUF
        + [pltpu.VMEM_SHARED((NPAD, HID), _f32)] * 2
        + [_DMA] * (2 * NBUF)
    ),
    compiler_params=_sc_params,
)


PK = 8
NR = N // PK


def _proj_body(x3_ref, wl_ref, wr_ref, p_ref, r_ref):
    x3 = x3_ref[...]
    wl = wl_ref[...]
    wr = wr_ref[...]
    pparts, rparts = [], []
    for b in range(PK):
        xb = x3[:, b, :]
        pparts.append(lax.dot_general(xb, wl, (((1,), (1,)), ((), ())),
                                      preferred_element_type=_f32))
        rparts.append(lax.dot_general(xb, wr, (((1,), (1,)), ((), ())),
                                      preferred_element_type=_f32))
    p_ref[...] = jnp.concatenate(pparts, axis=1)
    r_ref[...] = jnp.concatenate(rparts, axis=1)


def _mid_body(s_ref, c_ref, bl_ref, r_ref,
              wl_ref, wr_ref, p_out, r_out, inv_out):
    inv = 1.0 / jnp.maximum(c_ref[0] + c_ref[1], 1.0)
    h = jnp.maximum((s_ref[0] + s_ref[1]) * inv
                    + bl_ref[...] + r_ref[...], 0.0)
    p_out[...] = jnp.dot(h, wl_ref[...], preferred_element_type=_f32)
    r_out[...] = jnp.dot(h, wr_ref[...], preferred_element_type=_f32)
    inv_out[...] = inv


def _act_body(s_ref, inv_ref, bl_ref, r_ref, h_out):
    h_out[...] = jnp.maximum((s_ref[0] + s_ref[1]) * inv_ref[...]
                             + bl_ref[...] + r_ref[...], 0.0)


def _final_body(s_ref, inv_ref, h_ref, wl_ref, wr_ref, bl_ref, out_ref):
    mean = (s_ref[0] + s_ref[1]) * inv_ref[...]
    out_ref[...] = (jnp.dot(mean, wl_ref[...], preferred_element_type=_f32)
                    + jnp.dot(h_ref[...], wr_ref[...],
                              preferred_element_type=_f32)
                    + bl_ref[...])


_proj = pl.pallas_call(
    _proj_body,
    out_shape=(jax.ShapeDtypeStruct((NR, 128), _f32),
               jax.ShapeDtypeStruct((NR, 128), _f32)))

_mid = pl.pallas_call(
    _mid_body,
    out_shape=(jax.ShapeDtypeStruct((NR, 128), _f32),
               jax.ShapeDtypeStruct((NR, 128), _f32),
               jax.ShapeDtypeStruct((NR, 128), _f32)))

_act = pl.pallas_call(
    _act_body,
    out_shape=jax.ShapeDtypeStruct((NR, 128), _f32))

_final = pl.pallas_call(
    _final_body,
    out_shape=jax.ShapeDtypeStruct((NR, PK * OUT_CH), _f32))


def kernel(edge_index, features, Wl0, bl0, Wr0, Wl1, bl1, Wr1, Wl2, bl2, Wr2):
    edges_t = edge_index.astype(jnp.int32).reshape(2, NCH, CHUNK).transpose(
        1, 0, 2)
    zeros = jnp.zeros((SLAB, HID), _f32)
    ones = jnp.ones((CHUNK, HID), _f32)
    x3 = features.reshape(NR, PK, IN_CH)
    eye8 = jnp.eye(PK, dtype=_f32)
    bd1l = jnp.kron(eye8, Wl1.T)
    bd1r = jnp.kron(eye8, Wr1.T)
    ws2l = jnp.kron(eye8, Wl2.T)
    ws2r = jnp.kron(eye8, Wr2.T)
    blt0 = jnp.tile(bl0, PK).reshape(1, 128)
    blt1 = jnp.tile(bl1, PK).reshape(1, 128)
    blt2 = jnp.tile(bl2, PK).reshape(1, PK * OUT_CH)

    p0p, r0p = _proj(x3, Wl0, Wr0)
    s0, c0 = _seg_cnt(p0p.reshape(N, HID), edges_t, zeros, ones)
    p1p, r1p, invp = _mid(s0.reshape(NC, NR, 128), c0.reshape(NC, NR, 128),
                          blt0, r0p, bd1l, bd1r)
    s1 = _seg(p1p.reshape(N, HID), edges_t, zeros)
    h1p = _act(s1.reshape(NC, NR, 128), invp, blt1, r1p)
    s2 = _seg(h1p.reshape(N, HID), edges_t, zeros)
    outb = _final(s2.reshape(NC, NR, 128), invp, h1p, ws2l, ws2r, blt2)
    return outb.reshape(N, OUT_CH)
